# Initial kernel scaffold; baseline (speedup 1.0000x reference)
import jax
import jax.numpy as jnp
from jax.experimental import pallas as pl


def kernel(x, edge_index, edge_attr, W_self, W_neigh, b_sage, att, W1, b1, W2, b2):
    raise NotImplementedError("write your pallas kernel here")


# speedup vs baseline: 23.1506x; 23.1506x over previous
import functools

import jax
import jax.numpy as jnp
from jax import lax
from jax.experimental import pallas as pl
from jax.experimental.pallas import tpu as pltpu
from jax.experimental.pallas import tpu_sc as plsc

_NC, _NS = 2, 16
_NW = _NC * _NS
_CHUNK = 128
_LANES = 16


def _xalpha_body(xf_ref, amat_ref, out_ref):
    out_ref[...] = jnp.dot(xf_ref[...], amat_ref[...],
                           preferred_element_type=jnp.float32,
                           precision=jax.lax.Precision.HIGHEST)


def _head_body(xa_ref, pp_ref, dp_ref, wself_ref, wneigh_ref, bsage_ref,
               w1_ref, b1_ref, w2_ref, b2_ref, out_ref, hid_ref):
    agg = pp_ref[0] + pp_ref[1]
    deg = jnp.sum(dp_ref[...], axis=1, keepdims=True)
    neigh = agg / (deg + 1e-6)
    hmat = (jnp.dot(xa_ref[...], wself_ref[...],
                    preferred_element_type=jnp.float32)
            + jnp.dot(neigh, wneigh_ref[...],
                      preferred_element_type=jnp.float32)
            + bsage_ref[...])
    hid_ref[...] = hmat
    h1 = jnp.dot(jnp.maximum(hmat, 0.0), w1_ref[...],
                 preferred_element_type=jnp.float32) + b1_ref[...]
    out_ref[...] = jnp.dot(jnp.maximum(h1, 0.0), w2_ref[...],
                           preferred_element_type=jnp.float32) + b2_ref[...]


def _make_sc_scatter(n_pad, f, e):
    n_chunks = e // _CHUNK
    assert n_chunks % _NW == 0
    nfull = n_chunks // _NW
    rows_per_tile = n_pad // _NS
    copies = rows_per_tile // _CHUNK
    groups = f // _LANES
    mesh = plsc.VectorSubcoreMesh(core_axis_name="c", subcore_axis_name="s",
                                  num_cores=_NC, num_subcores=_NS)

    @functools.partial(
        pl.kernel,
        out_type=[jax.ShapeDtypeStruct((_NC, n_pad, f), jnp.float32),
                  jax.ShapeDtypeStruct((_NW, n_pad), jnp.float32)],
        mesh=mesh,
        compiler_params=pltpu.CompilerParams(needs_layout_passes=False),
        scratch_types=[
            pltpu.VMEM((_CHUNK,), jnp.int32),
            pltpu.VMEM((_CHUNK,), jnp.int32),
            pltpu.VMEM((_CHUNK,), jnp.float32),
            pltpu.VMEM((_CHUNK, f), jnp.float32),
            pltpu.VMEM((n_pad,), jnp.float32),
            pltpu.VMEM_SHARED((n_pad, f), jnp.float32),
            pltpu.SemaphoreType.DMA,
        ],
    )
    def sc_kernel(xa, src, dst, attr, outp, outd, srcv, dstv, attrv, rows,
                  degv, agg_sh, sem):
        cid = lax.axis_index("c")
        sid = lax.axis_index("s")
        wid = cid * _NS + sid
        lane_iota = lax.iota(jnp.int32, _LANES)

        def zero_row(i, carry):
            for g in range(groups):
                rows[i, pl.ds(g * _LANES, _LANES)] = jnp.zeros(
                    (_LANES,), jnp.float32)
            return carry
        lax.fori_loop(0, _CHUNK, zero_row, 0)

        def zero_deg(i, carry):
            degv[pl.ds(i * _LANES, _LANES)] = jnp.zeros((_LANES,),
                                                        jnp.float32)
            return carry
        lax.fori_loop(0, n_pad // _LANES, zero_deg, 0)

        row0 = sid * rows_per_tile
        for r in range(copies):
            pltpu.sync_copy(rows, agg_sh.at[pl.ds(row0 + r * _CHUNK, _CHUNK)])
        plsc.subcore_barrier()

        def chunk_body(k, carry):
            base = (k * _NW + wid) * _CHUNK
            pltpu.sync_copy(src.at[pl.ds(base, _CHUNK)], srcv)
            pltpu.sync_copy(dst.at[pl.ds(base, _CHUNK)], dstv)
            pltpu.sync_copy(attr.at[pl.ds(base, _CHUNK)], attrv)
            pltpu.async_copy(xa.at[srcv], rows, sem).wait()

            def scale_group(g2, c2):
                a16 = attrv[pl.ds(g2 * _LANES, _LANES)]
                d16 = dstv[pl.ds(g2 * _LANES, _LANES)]
                for j in range(_LANES):
                    ab = a16.at[jnp.full((_LANES,), j, jnp.int32)].get(
                        mode="promise_in_bounds")
                    i = g2 * _LANES + j
                    for g in range(groups):
                        sl = pl.ds(g * _LANES, _LANES)
                        rows[i, sl] = rows[i, sl] * ab
                    plsc.addupdate_scatter(degv, [d16], a16,
                                           mask=lane_iota == j)
                return c2
            lax.fori_loop(0, _CHUNK // _LANES, scale_group, 0)

            pltpu.sync_copy(rows, agg_sh.at[dstv], add=True)
            return carry
        lax.fori_loop(0, nfull, chunk_body, 0)

        pltpu.sync_copy(degv, outd.at[wid])
        plsc.subcore_barrier()
        for r in range(copies):
            sl = pl.ds(row0 + r * _CHUNK, _CHUNK)
            pltpu.sync_copy(agg_sh.at[sl], outp.at[cid, sl])

    return sc_kernel


def kernel(x, edge_index, edge_attr, W_self, W_neigh, b_sage, att, W1, b1,
           W2, b2):
    n, f, t = x.shape
    e = edge_attr.shape[0]
    hs = W_self.shape[1]
    hid = W1.shape[1]
    od = W2.shape[1]
    stripe = _NS * _CHUNK
    n_pad = ((n + stripe - 1) // stripe) * stripe
    bn = 1000
    assert n % bn == 0 and f % _LANES == 0

    alpha = jax.nn.softmax(att.astype(jnp.float32))
    amat = (jnp.eye(f, dtype=jnp.float32)[:, None, :]
            * alpha[None, :, None]).reshape(f * t, f)
    x_flat = x.reshape(n, f * t)
    eblk = _NW * _CHUNK
    e_pad = ((e + eblk - 1) // eblk) * eblk
    src = jnp.pad(edge_index[0].astype(jnp.int32), (0, e_pad - e))
    dst = jnp.pad(edge_index[1].astype(jnp.int32), (0, e_pad - e))
    attr = jnp.pad(edge_attr.astype(jnp.float32), (0, e_pad - e))

    x_alpha = pl.pallas_call(
        _xalpha_body,
        grid=(n // bn,),
        in_specs=[pl.BlockSpec((bn, f * t), lambda i: (i, 0)),
                  pl.BlockSpec((f * t, f), lambda i: (0, 0))],
        out_specs=pl.BlockSpec((bn, f), lambda i: (i, 0)),
        out_shape=jax.ShapeDtypeStruct((n, f), jnp.float32),
    )(x_flat, amat)

    partials, deg_partials = _make_sc_scatter(n_pad, f, e_pad)(
        x_alpha, src, dst, attr)
    deg_t = deg_partials.T

    out, hidden = pl.pallas_call(
        _head_body,
        grid=(n // bn,),
        in_specs=[
            pl.BlockSpec((bn, f), lambda i: (i, 0)),
            pl.BlockSpec((_NC, bn, f), lambda i: (0, i, 0)),
            pl.BlockSpec((bn, _NW), lambda i: (i, 0)),
            pl.BlockSpec((f, hs), lambda i: (0, 0)),
            pl.BlockSpec((f, hs), lambda i: (0, 0)),
            pl.BlockSpec((1, hs), lambda i: (0, 0)),
            pl.BlockSpec((hs, hid), lambda i: (0, 0)),
            pl.BlockSpec((1, hid), lambda i: (0, 0)),
            pl.BlockSpec((hid, od), lambda i: (0, 0)),
            pl.BlockSpec((1, od), lambda i: (0, 0)),
        ],
        out_specs=[pl.BlockSpec((bn, od), lambda i: (i, 0)),
                   pl.BlockSpec((bn, hs), lambda i: (i, 0))],
        out_shape=[jax.ShapeDtypeStruct((n, od), jnp.float32),
                   jax.ShapeDtypeStruct((n, hs), jnp.float32)],
    )(x_alpha, partials, deg_t, W_self, W_neigh,
      b_sage.reshape(1, hs), W1, b1.reshape(1, hid), W2, b2.reshape(1, od))
    return (out, hidden)
